# baseline (speedup 1.0000x reference)
import functools

import jax
import jax.numpy as jnp
from jax import lax
from jax.experimental import pallas as pl
from jax.experimental.pallas import tpu as pltpu
from jax.experimental.pallas import tpu_sc as plsc

GAMMA = 12.0
ADV_T = 1.0
NEG_RATIO = 16
DIM = 64
LANES = 16
CHUNK = 128


def _make_sc_partials(T, n_cores, n_subcores):
    nw = n_cores * n_subcores
    assert T % (nw * CHUNK) == 0, (T, nw, CHUNK)
    cpw = T // (nw * CHUNK)
    mesh = plsc.VectorSubcoreMesh(core_axis_name="c", subcore_axis_name="s")

    @functools.partial(
        pl.kernel,
        mesh=mesh,
        out_type=jax.ShapeDtypeStruct((T, LANES), jnp.float32),
        scratch_types=[
            pltpu.VMEM((CHUNK + LANES,), jnp.int32),
            pltpu.VMEM((CHUNK + LANES,), jnp.int32),
            pltpu.VMEM((CHUNK + LANES,), jnp.int32),
            pltpu.VMEM((CHUNK,), jnp.int32),
            pltpu.VMEM((CHUNK,), jnp.int32),
            pltpu.VMEM((CHUNK,), jnp.int32),
            pltpu.VMEM((CHUNK, 2 * DIM), jnp.float32),
            pltpu.VMEM((CHUNK, 2 * DIM), jnp.float32),
            pltpu.VMEM((CHUNK, 2 * DIM), jnp.float32),
            pltpu.VMEM((CHUNK, LANES), jnp.float32),
            pltpu.SemaphoreType.DMA,
        ],
    )
    def sc_partials(ent, rel, hidx, ridx, tidx, out, hv, rv, tv, gh, gr, gt,
                    hrows, rrows, trows, parts_v, sem):
        wid = lax.axis_index("s") * n_cores + lax.axis_index("c")

        def chunk_body(c, carry):
            base = (wid * cpw + c) * CHUNK
            pltpu.sync_copy(hidx.at[pl.ds(base, CHUNK)], hv.at[pl.ds(0, CHUNK)])
            pltpu.sync_copy(ridx.at[pl.ds(base, CHUNK)], rv.at[pl.ds(0, CHUNK)])
            pltpu.sync_copy(tidx.at[pl.ds(base, CHUNK)], tv.at[pl.ds(0, CHUNK)])

            def idx_body(k, carry2):
                s = pl.ds(k * LANES, LANES)
                gh[s] = hv[s] >> 1
                gr[s] = rv[s] >> 1
                gt[s] = tv[s] >> 1
                return carry2

            lax.fori_loop(0, CHUNK // LANES, idx_body, 0)
            d1 = pltpu.async_copy(ent.at[gh], hrows, sem)
            d2 = pltpu.async_copy(rel.at[gr], rrows, sem)
            d3 = pltpu.async_copy(ent.at[gt], trows, sem)
            d1.wait()
            d2.wait()
            d3.wait()

            def row_body(i, carry2):
                oh = (hv[pl.ds(i, LANES)][0] & 1) * DIM
                orr = (rv[pl.ds(i, LANES)][0] & 1) * DIM
                ot = (tv[pl.ds(i, LANES)][0] & 1) * DIM
                acc = jnp.zeros((LANES,), jnp.float32)
                for c2 in range(DIM // LANES):
                    hx = hrows[i, pl.ds(oh + c2 * LANES, LANES)]
                    rx = rrows[i, pl.ds(orr + c2 * LANES, LANES)]
                    tx = trows[i, pl.ds(ot + c2 * LANES, LANES)]
                    acc = acc + jnp.abs(hx + rx - tx)
                parts_v[i, :] = acc
                return carry2

            lax.fori_loop(0, CHUNK, row_body, 0)
            pltpu.sync_copy(parts_v, out.at[pl.ds(base, CHUNK)])
            return carry

        lax.fori_loop(0, cpw, chunk_body, 0)

    return sc_partials


def _loss_body(pos_ref, neg_ref, out_ref):
    pos_parts = pos_ref[...]
    neg_parts = neg_ref[...]

    def lsig(x):
        return jnp.minimum(x, 0.0) - jnp.log(1.0 + jnp.exp(-jnp.abs(x)))

    pos = GAMMA - jnp.sum(pos_parts, axis=1)
    ii = lax.broadcasted_iota(jnp.int32, (NEG_RATIO * LANES, NEG_RATIO), 0)
    jj = lax.broadcasted_iota(jnp.int32, (NEG_RATIO * LANES, NEG_RATIO), 1)
    fold = jnp.where(ii // LANES == jj, 1.0, 0.0).astype(jnp.float32)
    neg = GAMMA - jnp.dot(neg_parts, fold,
                          preferred_element_type=jnp.float32)

    z = neg * ADV_T
    m = jnp.max(z, axis=1, keepdims=True)
    e = jnp.exp(z - m)
    w = e / jnp.sum(e, axis=1, keepdims=True)
    neg_terms = jnp.sum(w * lsig(-neg), axis=1)
    pos_loss = -jnp.sum(lsig(pos)) / pos.shape[0]
    neg_loss = -jnp.sum(neg_terms) / neg.shape[0]
    out_ref[0, 0] = (pos_loss + neg_loss) * 0.5


def kernel(px, nx, py, ny, entity_embedding, relation_embedding):
    B = px.shape[0]
    N = nx.shape[0]
    T = B + N
    h_idx = jnp.concatenate([px[:, 0], nx[:, 0]])
    r_idx = jnp.concatenate([px[:, 1], nx[:, 1]])
    t_idx = jnp.concatenate([px[:, 2], nx[:, 2]])

    nent = entity_embedding.shape[0]
    ent2 = entity_embedding.reshape(nent // 2, 2 * DIM)
    rel2 = relation_embedding.reshape(nent // 2, 2 * DIM)

    info = plsc.get_sparse_core_info()
    sc_partials = _make_sc_partials(T, info.num_cores, info.num_subcores)
    parts = sc_partials(ent2, rel2, h_idx, r_idx, t_idx)

    pos_parts = parts[:B]
    neg_parts = parts[B:].reshape(B, NEG_RATIO * LANES)
    loss = pl.pallas_call(
        _loss_body,
        out_shape=jax.ShapeDtypeStruct((1, 1), jnp.float32),
        out_specs=pl.BlockSpec(memory_space=pltpu.SMEM),
    )(pos_parts, neg_parts)
    return loss[0, 0]


# speedup vs baseline: 1.0303x; 1.0303x over previous
import functools

import jax
import jax.numpy as jnp
from jax import lax
from jax.experimental import pallas as pl
from jax.experimental.pallas import tpu as pltpu
from jax.experimental.pallas import tpu_sc as plsc

GAMMA = 12.0
ADV_T = 1.0
NEG_RATIO = 16
DIM = 64
LANES = 16
CHUNK = 128


def _make_sc_partials(T, n_cores, n_subcores):
    nw = n_cores * n_subcores
    assert T % (nw * CHUNK) == 0, (T, nw, CHUNK)
    cpw = T // (nw * CHUNK)
    mesh = plsc.VectorSubcoreMesh(core_axis_name="c", subcore_axis_name="s")

    @functools.partial(
        pl.kernel,
        mesh=mesh,
        compiler_params=pltpu.CompilerParams(use_tc_tiling_on_sc=False),
        out_type=jax.ShapeDtypeStruct((T, LANES), jnp.float32),
        scratch_types=[
            pltpu.VMEM((CHUNK,), jnp.int32),
            pltpu.VMEM((CHUNK,), jnp.int32),
            pltpu.VMEM((CHUNK,), jnp.int32),
            pltpu.VMEM((CHUNK, DIM), jnp.float32),
            pltpu.VMEM((CHUNK, DIM), jnp.float32),
            pltpu.VMEM((CHUNK, DIM), jnp.float32),
            pltpu.VMEM((CHUNK, LANES), jnp.float32),
            pltpu.SemaphoreType.DMA,
        ],
    )
    def sc_partials(ent, rel, hidx, ridx, tidx, out, hv, rv, tv, hrows, rrows,
                    trows, parts_v, sem):
        wid = lax.axis_index("s") * n_cores + lax.axis_index("c")

        def chunk_body(c, carry):
            base = (wid * cpw + c) * CHUNK
            pltpu.sync_copy(hidx.at[pl.ds(base, CHUNK)], hv)
            pltpu.sync_copy(ridx.at[pl.ds(base, CHUNK)], rv)
            pltpu.sync_copy(tidx.at[pl.ds(base, CHUNK)], tv)
            d1 = pltpu.async_copy(ent.at[hv], hrows, sem)
            d2 = pltpu.async_copy(rel.at[rv], rrows, sem)
            d3 = pltpu.async_copy(ent.at[tv], trows, sem)
            d1.wait()
            d2.wait()
            d3.wait()

            def row_body(i, carry2):
                acc = jnp.zeros((LANES,), jnp.float32)
                for c2 in range(DIM // LANES):
                    hx = hrows[i, pl.ds(c2 * LANES, LANES)]
                    rx = rrows[i, pl.ds(c2 * LANES, LANES)]
                    tx = trows[i, pl.ds(c2 * LANES, LANES)]
                    acc = acc + jnp.abs(hx + rx - tx)
                parts_v[i, :] = acc
                return carry2

            lax.fori_loop(0, CHUNK, row_body, 0)
            pltpu.sync_copy(parts_v, out.at[pl.ds(base, CHUNK)])
            return carry

        lax.fori_loop(0, cpw, chunk_body, 0)

    return sc_partials


def _loss_body(pos_ref, neg_ref, out_ref):
    pos_parts = pos_ref[...]
    neg_parts = neg_ref[...]

    def lsig(x):
        return jnp.minimum(x, 0.0) - jnp.log(1.0 + jnp.exp(-jnp.abs(x)))

    pos = GAMMA - jnp.sum(pos_parts, axis=1)
    ii = lax.broadcasted_iota(jnp.int32, (NEG_RATIO * LANES, NEG_RATIO), 0)
    jj = lax.broadcasted_iota(jnp.int32, (NEG_RATIO * LANES, NEG_RATIO), 1)
    fold = jnp.where(ii // LANES == jj, 1.0, 0.0).astype(jnp.float32)
    neg = GAMMA - jnp.dot(neg_parts, fold,
                          preferred_element_type=jnp.float32)

    z = neg * ADV_T
    m = jnp.max(z, axis=1, keepdims=True)
    e = jnp.exp(z - m)
    w = e / jnp.sum(e, axis=1, keepdims=True)
    neg_terms = jnp.sum(w * lsig(-neg), axis=1)
    pos_loss = -jnp.sum(lsig(pos)) / pos.shape[0]
    neg_loss = -jnp.sum(neg_terms) / neg.shape[0]
    out_ref[0, 0] = (pos_loss + neg_loss) * 0.5


def kernel(px, nx, py, ny, entity_embedding, relation_embedding):
    B = px.shape[0]
    N = nx.shape[0]
    T = B + N
    h_idx = jnp.concatenate([px[:, 0], nx[:, 0]])
    r_idx = jnp.concatenate([px[:, 1], nx[:, 1]])
    t_idx = jnp.concatenate([px[:, 2], nx[:, 2]])

    info = plsc.get_sparse_core_info()
    sc_partials = _make_sc_partials(T, info.num_cores, info.num_subcores)
    parts = sc_partials(entity_embedding, relation_embedding, h_idx, r_idx,
                        t_idx)

    pos_parts = parts[:B]
    neg_parts = parts[B:].reshape(B, NEG_RATIO * LANES)
    loss = pl.pallas_call(
        _loss_body,
        out_shape=jax.ShapeDtypeStruct((1, 1), jnp.float32),
        out_specs=pl.BlockSpec(memory_space=pltpu.SMEM),
    )(pos_parts, neg_parts)
    return loss[0, 0]


# speedup vs baseline: 1.0846x; 1.0527x over previous
import functools

import jax
import jax.numpy as jnp
from jax import lax
from jax.experimental import pallas as pl
from jax.experimental.pallas import tpu as pltpu
from jax.experimental.pallas import tpu_sc as plsc

GAMMA = 12.0
ADV_T = 1.0
NEG_RATIO = 16
DIM = 64
LANES = 16
CHUNK = 128


def _make_sc_partials(B, N, n_cores, n_subcores):
    nw = n_cores * n_subcores
    assert B % (nw * CHUNK) == 0, (B, nw, CHUNK)
    assert N % (nw * CHUNK) == 0, (N, nw, CHUNK)
    pcw = B // (nw * CHUNK)
    ncw = N // (nw * CHUNK)
    nrows = CHUNK // NEG_RATIO
    mesh = plsc.VectorSubcoreMesh(core_axis_name="c", subcore_axis_name="s")

    @functools.partial(
        pl.kernel,
        mesh=mesh,
        compiler_params=pltpu.CompilerParams(use_tc_tiling_on_sc=False),
        out_type=[
            jax.ShapeDtypeStruct((B, LANES), jnp.float32),
            jax.ShapeDtypeStruct((B, NEG_RATIO * LANES), jnp.float32),
        ],
        scratch_types=[
            pltpu.VMEM((CHUNK,), jnp.int32),
            pltpu.VMEM((CHUNK,), jnp.int32),
            pltpu.VMEM((CHUNK,), jnp.int32),
            pltpu.VMEM((CHUNK, DIM), jnp.float32),
            pltpu.VMEM((CHUNK, DIM), jnp.float32),
            pltpu.VMEM((CHUNK, DIM), jnp.float32),
            pltpu.VMEM((CHUNK, LANES), jnp.float32),
            pltpu.VMEM((CHUNK // NEG_RATIO, NEG_RATIO * LANES), jnp.float32),
            pltpu.SemaphoreType.DMA,
        ],
    )
    def sc_partials(ent, rel, hidx, ridx, tidx, out_pos, out_neg, hv, rv, tv,
                    hrows, rrows, trows, pparts_v, nparts_v, sem):
        wid = lax.axis_index("s") * n_cores + lax.axis_index("c")

        def gather_chunk(base):
            pltpu.sync_copy(hidx.at[pl.ds(base, CHUNK)], hv)
            pltpu.sync_copy(ridx.at[pl.ds(base, CHUNK)], rv)
            pltpu.sync_copy(tidx.at[pl.ds(base, CHUNK)], tv)
            d1 = pltpu.async_copy(ent.at[hv], hrows, sem)
            d2 = pltpu.async_copy(rel.at[rv], rrows, sem)
            d3 = pltpu.async_copy(ent.at[tv], trows, sem)
            d1.wait()
            d2.wait()
            d3.wait()

        def row_acc(i):
            acc = jnp.zeros((LANES,), jnp.float32)
            for c2 in range(DIM // LANES):
                hx = hrows[i, pl.ds(c2 * LANES, LANES)]
                rx = rrows[i, pl.ds(c2 * LANES, LANES)]
                tx = trows[i, pl.ds(c2 * LANES, LANES)]
                acc = acc + jnp.abs(hx + rx - tx)
            return acc

        def pos_chunk(p, carry):
            base = (wid * pcw + p) * CHUNK
            gather_chunk(base)

            def row_body(i, carry2):
                pparts_v[i, :] = row_acc(i)
                return carry2

            lax.fori_loop(0, CHUNK, row_body, 0)
            pltpu.sync_copy(pparts_v, out_pos.at[pl.ds(base, CHUNK)])
            return carry

        lax.fori_loop(0, pcw, pos_chunk, 0)

        def neg_chunk(c, carry):
            nchunk = wid * ncw + c
            gather_chunk(B + nchunk * CHUNK)

            def row_body(i, carry2):
                nparts_v[i // NEG_RATIO,
                         pl.ds((i % NEG_RATIO) * LANES, LANES)] = row_acc(i)
                return carry2

            lax.fori_loop(0, CHUNK, row_body, 0)
            pltpu.sync_copy(nparts_v, out_neg.at[pl.ds(nchunk * nrows, nrows)])
            return carry

        lax.fori_loop(0, ncw, neg_chunk, 0)

    return sc_partials


def _loss_body(pos_ref, neg_ref, out_ref):
    pos_parts = pos_ref[...]
    neg_parts = neg_ref[...]

    def lsig(x):
        return jnp.minimum(x, 0.0) - jnp.log(1.0 + jnp.exp(-jnp.abs(x)))

    pos = GAMMA - jnp.sum(pos_parts, axis=1)
    ii = lax.broadcasted_iota(jnp.int32, (NEG_RATIO * LANES, NEG_RATIO), 0)
    jj = lax.broadcasted_iota(jnp.int32, (NEG_RATIO * LANES, NEG_RATIO), 1)
    fold = jnp.where(ii // LANES == jj, 1.0, 0.0).astype(jnp.float32)
    neg = GAMMA - jnp.dot(neg_parts, fold,
                          preferred_element_type=jnp.float32)

    z = neg * ADV_T
    m = jnp.max(z, axis=1, keepdims=True)
    e = jnp.exp(z - m)
    w = e / jnp.sum(e, axis=1, keepdims=True)
    neg_terms = jnp.sum(w * lsig(-neg), axis=1)
    pos_loss = -jnp.sum(lsig(pos)) / pos.shape[0]
    neg_loss = -jnp.sum(neg_terms) / neg.shape[0]
    out_ref[0, 0] = (pos_loss + neg_loss) * 0.5


def kernel(px, nx, py, ny, entity_embedding, relation_embedding):
    B = px.shape[0]
    N = nx.shape[0]
    T = B + N
    h_idx = jnp.concatenate([px[:, 0], nx[:, 0]])
    r_idx = jnp.concatenate([px[:, 1], nx[:, 1]])
    t_idx = jnp.concatenate([px[:, 2], nx[:, 2]])

    info = plsc.get_sparse_core_info()
    sc_partials = _make_sc_partials(B, N, info.num_cores, info.num_subcores)
    pos_parts, neg_parts = sc_partials(entity_embedding, relation_embedding,
                                       h_idx, r_idx, t_idx)

    loss = pl.pallas_call(
        _loss_body,
        out_shape=jax.ShapeDtypeStruct((1, 1), jnp.float32),
        out_specs=pl.BlockSpec(memory_space=pltpu.SMEM),
    )(pos_parts, neg_parts)
    return loss[0, 0]
